# baseline (speedup 1.0000x reference)
import functools

import jax
import jax.numpy as jnp
from jax import lax
from jax.experimental import pallas as pl
from jax.experimental.pallas import tpu as pltpu
from jax.experimental.pallas import tpu_sc as plsc

F32 = jnp.float32

_N = 10000
_E = 320000
_NW = 32
_EPT = _E // _NW
_CH = 80
_NCH = _EPT // _CH
_ROWS_PER_TILE = _N // 16
_D = 16
_MD = 64
_BLK = 6400
_NBLK = _E // _BLK


def _sc_mesh():
    return plsc.VectorSubcoreMesh(core_axis_name="c", subcore_axis_name="s")


def _gather_body(ridx_hbm, sidx_hbm, xpad_hbm, xr_hbm, xs_hbm,
                 ridx_v, sidx_v, bufr0, bufs0, bufr1, bufs1, sem0, sem1):
    c = lax.axis_index("c")
    s = lax.axis_index("s")
    wid = s * 2 + c
    pltpu.sync_copy(ridx_hbm.at[wid], ridx_v)
    pltpu.sync_copy(sidx_hbm.at[wid], sidx_v)
    base = wid * _NCH

    def issue(j, br, bs, sem):
        pltpu.async_copy(xpad_hbm.at[ridx_v.at[j]], br, sem)
        pltpu.async_copy(xpad_hbm.at[sidx_v.at[j]], bs, sem)

    def wait_pair(br, bs, sem):
        pltpu.make_async_copy(xpad_hbm.at[pl.ds(0, _CH)], br, sem).wait()
        pltpu.make_async_copy(xpad_hbm.at[pl.ds(0, _CH)], bs, sem).wait()

    def store(j, br, bs):
        pltpu.sync_copy(br, xr_hbm.at[base + j, pl.ds(0, _CH)])
        pltpu.sync_copy(bs, xs_hbm.at[base + j, pl.ds(0, _CH)])

    issue(0, bufr0, bufs0, sem0)

    def body(i, carry):
        issue(2 * i + 1, bufr1, bufs1, sem1)
        wait_pair(bufr0, bufs0, sem0)
        store(2 * i, bufr0, bufs0)

        @pl.when(i < (_NCH - 1) // 2 - 1)
        def _():
            issue(2 * i + 2, bufr0, bufs0, sem0)

        wait_pair(bufr1, bufs1, sem1)
        store(2 * i + 1, bufr1, bufs1)
        return carry

    lax.fori_loop(0, (_NCH - 1) // 2, body, 0)
    issue(_NCH - 1, bufr0, bufs0, sem0)
    wait_pair(bufr0, bufs0, sem0)
    store(_NCH - 1, bufr0, bufs0)


def _sc_gather(ridx3, sidx3, xpad):
    run = functools.partial(
        pl.kernel,
        out_type=(jax.ShapeDtypeStruct((_E // _CH, _CH, _D), F32),
                  jax.ShapeDtypeStruct((_E // _CH, _CH, _D), F32)),
        mesh=_sc_mesh(),
        scratch_types=[
            pltpu.VMEM((_NCH, _CH), jnp.int32),
            pltpu.VMEM((_NCH, _CH), jnp.int32),
            pltpu.VMEM((_CH, _D), F32),
            pltpu.VMEM((_CH, _D), F32),
            pltpu.VMEM((_CH, _D), F32),
            pltpu.VMEM((_CH, _D), F32),
            pltpu.SemaphoreType.DMA,
            pltpu.SemaphoreType.DMA,
        ],
        compiler_params=pltpu.CompilerParams(use_tc_tiling_on_sc=False),
    )(_gather_body)
    return run(ridx3, sidx3, xpad)


def _scatter_body(ridx_hbm, msg_hbm, zeros_hbm, out_hbm,
                  ridx_v, mbuf0, mbuf1, zbuf, acc, sem0, sem1):
    c = lax.axis_index("c")
    s = lax.axis_index("s")
    wid = s * 2 + c
    pltpu.sync_copy(zeros_hbm, zbuf)

    def zbody(k, carry):
        pltpu.sync_copy(zbuf, acc.at[pl.ds(s * _ROWS_PER_TILE + k * 125, 125)])
        return carry

    lax.fori_loop(0, _ROWS_PER_TILE // 125, zbody, 0)
    pltpu.sync_copy(ridx_hbm.at[wid], ridx_v)
    plsc.subcore_barrier()
    base = wid * _NCH

    def issue(j, mb, sem):
        pltpu.async_copy(msg_hbm.at[base + j, pl.ds(0, _CH)], mb, sem)

    def wait(mb, sem):
        pltpu.make_async_copy(msg_hbm.at[0, pl.ds(0, _CH)], mb, sem).wait()

    def scat(j, mb):
        pltpu.sync_copy(mb, acc.at[ridx_v.at[j]], add=True)

    issue(0, mbuf0, sem0)

    def body(i, carry):
        issue(2 * i + 1, mbuf1, sem1)
        wait(mbuf0, sem0)
        scat(2 * i, mbuf0)

        @pl.when(i < (_NCH - 1) // 2 - 1)
        def _():
            issue(2 * i + 2, mbuf0, sem0)

        wait(mbuf1, sem1)
        scat(2 * i + 1, mbuf1)
        return carry

    lax.fori_loop(0, (_NCH - 1) // 2, body, 0)
    issue(_NCH - 1, mbuf0, sem0)
    wait(mbuf0, sem0)
    scat(_NCH - 1, mbuf0)
    plsc.subcore_barrier()

    @pl.when(s < 10)
    def _drain():
        def dbody(k, carry):
            pltpu.sync_copy(acc.at[pl.ds(s * 1000 + k * 125, 125)], zbuf)
            pltpu.sync_copy(zbuf, out_hbm.at[c * 10 + s, pl.ds(k * 125, 125)])
            return carry

        lax.fori_loop(0, 8, dbody, 0)


def _sc_scatter(ridx3, msg, zeros):
    run = functools.partial(
        pl.kernel,
        out_type=jax.ShapeDtypeStruct((20, 1000, _MD), F32),
        mesh=_sc_mesh(),
        scratch_types=[
            pltpu.VMEM((_NCH, _CH), jnp.int32),
            pltpu.VMEM((_CH, _MD), F32),
            pltpu.VMEM((_CH, _MD), F32),
            pltpu.VMEM((125, _MD), F32),
            pltpu.VMEM_SHARED((_N, _MD), F32),
            pltpu.SemaphoreType.DMA,
            pltpu.SemaphoreType.DMA,
        ],
        compiler_params=pltpu.CompilerParams(use_tc_tiling_on_sc=False),
    )(_scatter_body)
    return run(ridx3, msg, zeros)


def _edge_mlp_body(xr_ref, xs_ref, w0r_ref, w0s_ref, b0_ref,
                   w1_ref, b1_ref, w2_ref, b2_ref, w3_ref, b3_ref,
                   w4_ref, b4_ref, w5_ref, b5_ref, out_ref):
    bf = jnp.bfloat16

    def dotb(a, w):
        return jnp.dot(a, w, preferred_element_type=F32)

    pr = xr_ref[...]
    ps = xs_ref[...]
    w0r = w0r_ref[...]
    w0s = w0s_ref[...]
    parts = []
    for k in range(8):
        hk = (dotb(pr[:, 16 * k:16 * (k + 1)], w0r)
              + dotb(ps[:, 16 * k:16 * (k + 1)], w0s))
        parts.append(hk)
    h = jnp.concatenate(parts, axis=0)
    h = jnp.maximum(h + b0_ref[...], 0.0)
    h = jnp.maximum(dotb(h, w1_ref[...]) + b1_ref[...], 0.0)
    h = jnp.maximum(dotb(h, w2_ref[...]) + b2_ref[...], 0.0)
    h = jnp.maximum(dotb(h, w3_ref[...]) + b3_ref[...], 0.0)
    h = jnp.maximum(dotb(h, w4_ref[...]) + b4_ref[...], 0.0)
    msg = dotb(h, w5_ref[...]) + b5_ref[...]
    half = _BLK // 2
    out_ref[...] = jnp.concatenate([msg[:half], msg[half:]], axis=1)


def _edge_mlp(xr, xs, w0r, w0s, b0, w1, b1, w2, b2, w3, b3, w4, b4, w5, b5):
    rblk = _BLK * _D // 128
    oblk = _BLK * _MD // 128
    full = lambda shape: pl.BlockSpec(shape, lambda i: (0, 0))
    return pl.pallas_call(
        _edge_mlp_body,
        grid=(_NBLK,),
        in_specs=[
            pl.BlockSpec((rblk, 128), lambda i: (i, 0)),
            pl.BlockSpec((rblk, 128), lambda i: (i, 0)),
            full(w0r.shape), full(w0s.shape), full(b0.shape),
            full(w1.shape), full(b1.shape), full(w2.shape), full(b2.shape),
            full(w3.shape), full(b3.shape), full(w4.shape), full(b4.shape),
            full(w5.shape), full(b5.shape),
        ],
        out_specs=pl.BlockSpec((oblk, 128), lambda i: (i, 0)),
        out_shape=jax.ShapeDtypeStruct((_E * _MD // 128, 128), F32),
        compiler_params=pltpu.CompilerParams(
            dimension_semantics=("arbitrary",)),
    )(xr, xs, w0r, w0s, b0, w1, b1, w2, b2, w3, b3, w4, b4, w5, b5)


def _node_mlp_body(x2_ref, parts_ref, w0x_ref, w0a_ref, b0_ref,
                   w1_ref, b1_ref, w2_ref, b2_ref, out_ref):
    dot = functools.partial(jnp.dot, preferred_element_type=F32)
    half = _N // 2
    aggr = parts_ref[0:half, :] + parts_ref[half:_N, :]
    h = dot(x2_ref[...], w0x_ref[...]) + dot(aggr, w0a_ref[...])
    h = jnp.maximum(h + b0_ref[...], 0.0)
    h = jnp.maximum(dot(h, w1_ref[...]) + b1_ref[...], 0.0)
    out_ref[...] = dot(h, w2_ref[...]) + b2_ref[...]


def _node_mlp(x2, parts, w0x, w0a, b0, w1, b1, w2, b2):
    return pl.pallas_call(
        _node_mlp_body,
        out_shape=jax.ShapeDtypeStruct((_N // 2, 4), F32),
    )(x2, parts, w0x, w0a, b0, w1, b1, w2, b2)


def kernel(x, edge_index, edge_attr,
           e_w0, e_b0, e_w1, e_b1, e_w2, e_b2, e_w3, e_b3, e_w4, e_b4,
           e_w5, e_b5, n_w0, n_b0, n_w1, n_b1, n_w2, n_b2):
    del edge_attr
    sender = edge_index[0]
    receiver = edge_index[1]

    m_of_t = jnp.arange(_BLK).reshape(2, _BLK // 2).T.reshape(-1)
    sigma = 8 * (m_of_t % (_BLK // 8)) + m_of_t // (_BLK // 8)

    ridx_g = receiver.reshape(_NW, _NCH, _CH)
    sidx_g = sender.reshape(_NW, _NCH, _CH)

    xpad = jnp.zeros((_N, _D), F32).at[:, :5].set(x)

    a, b, cmat = e_w0[:, 0:5], e_w0[:, 5:10], e_w0[:, 10:15]
    w0r = jnp.zeros((_D, 150), F32).at[0:5, :].set((a + cmat).T)
    w0s = jnp.zeros((_D, 150), F32).at[0:5, :].set((b - cmat).T)
    w5 = jnp.zeros((150, _MD), F32).at[:, 0:50].set(e_w5.T)
    b5 = jnp.zeros((1, _MD), F32).at[:, 0:50].set(e_b5)

    xr, xs = _sc_gather(ridx_g, sidx_g, xpad)
    xr = xr.reshape(_E * _D // 128, 128)
    xs = xs.reshape(_E * _D // 128, 128)
    tok = lax.convert_element_type(xr[0, 0] * 0.0, jnp.int32)
    ridx_t = (receiver.reshape(_NBLK, _BLK)[:, sigma]
              .reshape(_NW, _NCH, _CH)) + tok
    msg = _edge_mlp(xr, xs, w0r, w0s,
                    e_b0.reshape(1, -1),
                    e_w1.T, e_b1.reshape(1, -1),
                    e_w2.T, e_b2.reshape(1, -1),
                    e_w3.T, e_b3.reshape(1, -1),
                    e_w4.T, e_b4.reshape(1, -1),
                    w5, b5)

    zeros = jnp.zeros((125, _MD), F32)
    partials = _sc_scatter(ridx_t, msg.reshape(_E // _CH, _CH, _MD), zeros)
    partials = partials.reshape(_N, 128)

    def blockdiag2(w):
        r, c = w.shape
        return (jnp.zeros((2 * r, 2 * c), F32)
                .at[0:r, 0:c].set(w).at[r:2 * r, c:2 * c].set(w))

    w0x = n_w0[:, 0:5].T
    w0a = jnp.zeros((_MD, 100), F32).at[0:50, :].set(n_w0[:, 5:55].T)
    out2 = _node_mlp(x.reshape(_N // 2, 10), partials,
                     blockdiag2(w0x), blockdiag2(w0a),
                     jnp.tile(n_b0, 2).reshape(1, -1),
                     blockdiag2(n_w1.T), jnp.tile(n_b1, 2).reshape(1, -1),
                     blockdiag2(n_w2.T), jnp.tile(n_b2, 2).reshape(1, -1))
    return out2.reshape(_N, 2)


# speedup vs baseline: 1.0045x; 1.0045x over previous
import functools

import jax
import jax.numpy as jnp
from jax import lax
from jax.experimental import pallas as pl
from jax.experimental.pallas import tpu as pltpu
from jax.experimental.pallas import tpu_sc as plsc

F32 = jnp.float32

_N = 10000
_E = 320000
_NW = 32
_EPT = _E // _NW
_CH = 80
_NCH = _EPT // _CH
_ROWS_PER_TILE = _N // 16
_D = 16
_MD = 64
_BLK = 12800
_NBLK = _E // _BLK


def _sc_mesh():
    return plsc.VectorSubcoreMesh(core_axis_name="c", subcore_axis_name="s")


def _gather_body(ridx_hbm, sidx_hbm, xpad_hbm, xr_hbm, xs_hbm,
                 ridx_v, sidx_v, bufr0, bufs0, bufr1, bufs1, sem0, sem1):
    c = lax.axis_index("c")
    s = lax.axis_index("s")
    wid = s * 2 + c
    pltpu.sync_copy(ridx_hbm.at[wid], ridx_v)
    pltpu.sync_copy(sidx_hbm.at[wid], sidx_v)
    base = wid * _NCH

    def issue(j, br, bs, sem):
        pltpu.async_copy(xpad_hbm.at[ridx_v.at[j]], br, sem)
        pltpu.async_copy(xpad_hbm.at[sidx_v.at[j]], bs, sem)

    def wait_pair(br, bs, sem):
        pltpu.make_async_copy(xpad_hbm.at[pl.ds(0, _CH)], br, sem).wait()
        pltpu.make_async_copy(xpad_hbm.at[pl.ds(0, _CH)], bs, sem).wait()

    def store(j, br, bs):
        pltpu.sync_copy(br, xr_hbm.at[base + j, pl.ds(0, _CH)])
        pltpu.sync_copy(bs, xs_hbm.at[base + j, pl.ds(0, _CH)])

    issue(0, bufr0, bufs0, sem0)

    def body(i, carry):
        issue(2 * i + 1, bufr1, bufs1, sem1)
        wait_pair(bufr0, bufs0, sem0)
        store(2 * i, bufr0, bufs0)

        @pl.when(i < (_NCH - 1) // 2 - 1)
        def _():
            issue(2 * i + 2, bufr0, bufs0, sem0)

        wait_pair(bufr1, bufs1, sem1)
        store(2 * i + 1, bufr1, bufs1)
        return carry

    lax.fori_loop(0, (_NCH - 1) // 2, body, 0)
    issue(_NCH - 1, bufr0, bufs0, sem0)
    wait_pair(bufr0, bufs0, sem0)
    store(_NCH - 1, bufr0, bufs0)


def _sc_gather(ridx3, sidx3, xpad):
    run = functools.partial(
        pl.kernel,
        out_type=(jax.ShapeDtypeStruct((_E // _CH, _CH, _D), F32),
                  jax.ShapeDtypeStruct((_E // _CH, _CH, _D), F32)),
        mesh=_sc_mesh(),
        scratch_types=[
            pltpu.VMEM((_NCH, _CH), jnp.int32),
            pltpu.VMEM((_NCH, _CH), jnp.int32),
            pltpu.VMEM((_CH, _D), F32),
            pltpu.VMEM((_CH, _D), F32),
            pltpu.VMEM((_CH, _D), F32),
            pltpu.VMEM((_CH, _D), F32),
            pltpu.SemaphoreType.DMA,
            pltpu.SemaphoreType.DMA,
        ],
        compiler_params=pltpu.CompilerParams(use_tc_tiling_on_sc=False),
    )(_gather_body)
    return run(ridx3, sidx3, xpad)


def _scatter_body(ridx_hbm, msg_hbm, zeros_hbm, out_hbm,
                  ridx_v, mbuf0, mbuf1, zbuf, acc, sem0, sem1):
    c = lax.axis_index("c")
    s = lax.axis_index("s")
    wid = s * 2 + c
    pltpu.sync_copy(zeros_hbm, zbuf)

    def zbody(k, carry):
        pltpu.sync_copy(zbuf, acc.at[pl.ds(s * _ROWS_PER_TILE + k * 125, 125)])
        return carry

    lax.fori_loop(0, _ROWS_PER_TILE // 125, zbody, 0)
    pltpu.sync_copy(ridx_hbm.at[wid], ridx_v)
    plsc.subcore_barrier()
    base = wid * _NCH

    def issue(j, mb, sem):
        pltpu.async_copy(msg_hbm.at[base + j, pl.ds(0, _CH)], mb, sem)

    def wait(mb, sem):
        pltpu.make_async_copy(msg_hbm.at[0, pl.ds(0, _CH)], mb, sem).wait()

    def scat(j, mb):
        pltpu.sync_copy(mb, acc.at[ridx_v.at[j]], add=True)

    issue(0, mbuf0, sem0)

    def body(i, carry):
        issue(2 * i + 1, mbuf1, sem1)
        wait(mbuf0, sem0)
        scat(2 * i, mbuf0)

        @pl.when(i < (_NCH - 1) // 2 - 1)
        def _():
            issue(2 * i + 2, mbuf0, sem0)

        wait(mbuf1, sem1)
        scat(2 * i + 1, mbuf1)
        return carry

    lax.fori_loop(0, (_NCH - 1) // 2, body, 0)
    issue(_NCH - 1, mbuf0, sem0)
    wait(mbuf0, sem0)
    scat(_NCH - 1, mbuf0)
    plsc.subcore_barrier()

    @pl.when(s < 10)
    def _drain():
        def dbody(k, carry):
            pltpu.sync_copy(acc.at[pl.ds(s * 1000 + k * 125, 125)], zbuf)
            pltpu.sync_copy(zbuf, out_hbm.at[c * 10 + s, pl.ds(k * 125, 125)])
            return carry

        lax.fori_loop(0, 8, dbody, 0)


def _sc_scatter(ridx3, msg, zeros):
    run = functools.partial(
        pl.kernel,
        out_type=jax.ShapeDtypeStruct((20, 1000, _MD), F32),
        mesh=_sc_mesh(),
        scratch_types=[
            pltpu.VMEM((_NCH, _CH), jnp.int32),
            pltpu.VMEM((_CH, _MD), F32),
            pltpu.VMEM((_CH, _MD), F32),
            pltpu.VMEM((125, _MD), F32),
            pltpu.VMEM_SHARED((_N, _MD), F32),
            pltpu.SemaphoreType.DMA,
            pltpu.SemaphoreType.DMA,
        ],
        compiler_params=pltpu.CompilerParams(use_tc_tiling_on_sc=False),
    )(_scatter_body)
    return run(ridx3, msg, zeros)


def _edge_mlp_body(xr_ref, xs_ref, w0r_ref, w0s_ref, b0_ref,
                   w1_ref, b1_ref, w2_ref, b2_ref, w3_ref, b3_ref,
                   w4_ref, b4_ref, w5_ref, b5_ref, out_ref):
    bf = jnp.bfloat16

    def dotb(a, w):
        return jnp.dot(a, w, preferred_element_type=F32)

    pr = xr_ref[...]
    ps = xs_ref[...]
    w0r = w0r_ref[...]
    w0s = w0s_ref[...]
    parts = []
    for k in range(8):
        hk = (dotb(pr[:, 16 * k:16 * (k + 1)], w0r)
              + dotb(ps[:, 16 * k:16 * (k + 1)], w0s))
        parts.append(hk)
    h = jnp.concatenate(parts, axis=0)
    h = jnp.maximum(h + b0_ref[...], 0.0)
    h = jnp.maximum(dotb(h, w1_ref[...]) + b1_ref[...], 0.0)
    h = jnp.maximum(dotb(h, w2_ref[...]) + b2_ref[...], 0.0)
    h = jnp.maximum(dotb(h, w3_ref[...]) + b3_ref[...], 0.0)
    h = jnp.maximum(dotb(h, w4_ref[...]) + b4_ref[...], 0.0)
    msg = dotb(h, w5_ref[...]) + b5_ref[...]
    half = _BLK // 2
    out_ref[...] = jnp.concatenate([msg[:half], msg[half:]], axis=1)


def _edge_mlp(xr, xs, w0r, w0s, b0, w1, b1, w2, b2, w3, b3, w4, b4, w5, b5):
    rblk = _BLK * _D // 128
    oblk = _BLK * _MD // 128
    full = lambda shape: pl.BlockSpec(shape, lambda i: (0, 0))
    return pl.pallas_call(
        _edge_mlp_body,
        grid=(_NBLK,),
        in_specs=[
            pl.BlockSpec((rblk, 128), lambda i: (i, 0)),
            pl.BlockSpec((rblk, 128), lambda i: (i, 0)),
            full(w0r.shape), full(w0s.shape), full(b0.shape),
            full(w1.shape), full(b1.shape), full(w2.shape), full(b2.shape),
            full(w3.shape), full(b3.shape), full(w4.shape), full(b4.shape),
            full(w5.shape), full(b5.shape),
        ],
        out_specs=pl.BlockSpec((oblk, 128), lambda i: (i, 0)),
        out_shape=jax.ShapeDtypeStruct((_E * _MD // 128, 128), F32),
        compiler_params=pltpu.CompilerParams(
            dimension_semantics=("arbitrary",)),
    )(xr, xs, w0r, w0s, b0, w1, b1, w2, b2, w3, b3, w4, b4, w5, b5)


def _node_mlp_body(x2_ref, parts_ref, w0x_ref, w0a_ref, b0_ref,
                   w1_ref, b1_ref, w2_ref, b2_ref, out_ref):
    dot = functools.partial(jnp.dot, preferred_element_type=F32)
    half = _N // 2
    aggr = parts_ref[0:half, :] + parts_ref[half:_N, :]
    h = dot(x2_ref[...], w0x_ref[...]) + dot(aggr, w0a_ref[...])
    h = jnp.maximum(h + b0_ref[...], 0.0)
    h = jnp.maximum(dot(h, w1_ref[...]) + b1_ref[...], 0.0)
    out_ref[...] = dot(h, w2_ref[...]) + b2_ref[...]


def _node_mlp(x2, parts, w0x, w0a, b0, w1, b1, w2, b2):
    return pl.pallas_call(
        _node_mlp_body,
        out_shape=jax.ShapeDtypeStruct((_N // 2, 4), F32),
    )(x2, parts, w0x, w0a, b0, w1, b1, w2, b2)


def kernel(x, edge_index, edge_attr,
           e_w0, e_b0, e_w1, e_b1, e_w2, e_b2, e_w3, e_b3, e_w4, e_b4,
           e_w5, e_b5, n_w0, n_b0, n_w1, n_b1, n_w2, n_b2):
    del edge_attr
    sender = edge_index[0]
    receiver = edge_index[1]

    m_of_t = jnp.arange(_BLK).reshape(2, _BLK // 2).T.reshape(-1)
    sigma = 8 * (m_of_t % (_BLK // 8)) + m_of_t // (_BLK // 8)

    ridx_g = receiver.reshape(_NW, _NCH, _CH)
    sidx_g = sender.reshape(_NW, _NCH, _CH)

    xpad = jnp.zeros((_N, _D), F32).at[:, :5].set(x)

    a, b, cmat = e_w0[:, 0:5], e_w0[:, 5:10], e_w0[:, 10:15]
    w0r = jnp.zeros((_D, 150), F32).at[0:5, :].set((a + cmat).T)
    w0s = jnp.zeros((_D, 150), F32).at[0:5, :].set((b - cmat).T)
    w5 = jnp.zeros((150, _MD), F32).at[:, 0:50].set(e_w5.T)
    b5 = jnp.zeros((1, _MD), F32).at[:, 0:50].set(e_b5)

    xr, xs = _sc_gather(ridx_g, sidx_g, xpad)
    xr = xr.reshape(_E * _D // 128, 128)
    xs = xs.reshape(_E * _D // 128, 128)
    tok = lax.convert_element_type(xr[0, 0] * 0.0, jnp.int32)
    ridx_t = (receiver.reshape(_NBLK, _BLK)[:, sigma]
              .reshape(_NW, _NCH, _CH)) + tok
    msg = _edge_mlp(xr, xs, w0r, w0s,
                    e_b0.reshape(1, -1),
                    e_w1.T, e_b1.reshape(1, -1),
                    e_w2.T, e_b2.reshape(1, -1),
                    e_w3.T, e_b3.reshape(1, -1),
                    e_w4.T, e_b4.reshape(1, -1),
                    w5, b5)

    zeros = jnp.zeros((125, _MD), F32)
    partials = _sc_scatter(ridx_t, msg.reshape(_E // _CH, _CH, _MD), zeros)
    partials = partials.reshape(_N, 128)

    def blockdiag2(w):
        r, c = w.shape
        return (jnp.zeros((2 * r, 2 * c), F32)
                .at[0:r, 0:c].set(w).at[r:2 * r, c:2 * c].set(w))

    w0x = n_w0[:, 0:5].T
    w0a = jnp.zeros((_MD, 100), F32).at[0:50, :].set(n_w0[:, 5:55].T)
    out2 = _node_mlp(x.reshape(_N // 2, 10), partials,
                     blockdiag2(w0x), blockdiag2(w0a),
                     jnp.tile(n_b0, 2).reshape(1, -1),
                     blockdiag2(n_w1.T), jnp.tile(n_b1, 2).reshape(1, -1),
                     blockdiag2(n_w2.T), jnp.tile(n_b2, 2).reshape(1, -1))
    return out2.reshape(_N, 2)


# speedup vs baseline: 1.0231x; 1.0185x over previous
import functools

import jax
import jax.numpy as jnp
from jax import lax
from jax.experimental import pallas as pl
from jax.experimental.pallas import tpu as pltpu
from jax.experimental.pallas import tpu_sc as plsc

F32 = jnp.float32

_N = 10000
_E = 320000
_NW = 32
_EPT = _E // _NW
_CH = 125
_NCH = _EPT // _CH
_ROWS_PER_TILE = _N // 16
_D = 16
_MD = 64
_BLK = 12800
_NBLK = _E // _BLK


def _sc_mesh():
    return plsc.VectorSubcoreMesh(core_axis_name="c", subcore_axis_name="s")


def _gather_body(ridx_hbm, sidx_hbm, xpad_hbm, xr_hbm, xs_hbm,
                 ridx_v, sidx_v, bufr0, bufs0, bufr1, bufs1, sem0, sem1):
    c = lax.axis_index("c")
    s = lax.axis_index("s")
    wid = s * 2 + c
    pltpu.sync_copy(ridx_hbm.at[wid], ridx_v)
    pltpu.sync_copy(sidx_hbm.at[wid], sidx_v)
    base = wid * _NCH

    def issue(j, br, bs, sem):
        pltpu.async_copy(xpad_hbm.at[ridx_v.at[j]], br, sem)
        pltpu.async_copy(xpad_hbm.at[sidx_v.at[j]], bs, sem)

    def wait_pair(br, bs, sem):
        pltpu.make_async_copy(xpad_hbm.at[pl.ds(0, _CH)], br, sem).wait()
        pltpu.make_async_copy(xpad_hbm.at[pl.ds(0, _CH)], bs, sem).wait()

    def store(j, br, bs):
        pltpu.sync_copy(br, xr_hbm.at[base + j, pl.ds(0, _CH)])
        pltpu.sync_copy(bs, xs_hbm.at[base + j, pl.ds(0, _CH)])

    issue(0, bufr0, bufs0, sem0)

    def body(i, carry):
        issue(2 * i + 1, bufr1, bufs1, sem1)
        wait_pair(bufr0, bufs0, sem0)
        store(2 * i, bufr0, bufs0)

        @pl.when(i < (_NCH - 1) // 2 - 1)
        def _():
            issue(2 * i + 2, bufr0, bufs0, sem0)

        wait_pair(bufr1, bufs1, sem1)
        store(2 * i + 1, bufr1, bufs1)
        return carry

    lax.fori_loop(0, (_NCH - 1) // 2, body, 0)
    issue(_NCH - 1, bufr0, bufs0, sem0)
    wait_pair(bufr0, bufs0, sem0)
    store(_NCH - 1, bufr0, bufs0)


def _sc_gather(ridx3, sidx3, xpad):
    run = functools.partial(
        pl.kernel,
        out_type=(jax.ShapeDtypeStruct((_E // _CH, _CH, _D), F32),
                  jax.ShapeDtypeStruct((_E // _CH, _CH, _D), F32)),
        mesh=_sc_mesh(),
        scratch_types=[
            pltpu.VMEM((_NCH, _CH), jnp.int32),
            pltpu.VMEM((_NCH, _CH), jnp.int32),
            pltpu.VMEM((_CH, _D), F32),
            pltpu.VMEM((_CH, _D), F32),
            pltpu.VMEM((_CH, _D), F32),
            pltpu.VMEM((_CH, _D), F32),
            pltpu.SemaphoreType.DMA,
            pltpu.SemaphoreType.DMA,
        ],
        compiler_params=pltpu.CompilerParams(use_tc_tiling_on_sc=False),
    )(_gather_body)
    return run(ridx3, sidx3, xpad)


def _scatter_body(ridx_hbm, msg_hbm, zeros_hbm, out_hbm,
                  ridx_v, mbuf0, mbuf1, zbuf, acc, sem0, sem1):
    c = lax.axis_index("c")
    s = lax.axis_index("s")
    wid = s * 2 + c
    pltpu.sync_copy(zeros_hbm, zbuf)

    def zbody(k, carry):
        pltpu.sync_copy(zbuf, acc.at[pl.ds(s * _ROWS_PER_TILE + k * 125, 125)])
        return carry

    lax.fori_loop(0, _ROWS_PER_TILE // 125, zbody, 0)
    pltpu.sync_copy(ridx_hbm.at[wid], ridx_v)
    plsc.subcore_barrier()
    base = wid * _NCH

    def issue(j, mb, sem):
        pltpu.async_copy(msg_hbm.at[base + j, pl.ds(0, _CH)], mb, sem)

    def wait(mb, sem):
        pltpu.make_async_copy(msg_hbm.at[0, pl.ds(0, _CH)], mb, sem).wait()

    def scat(j, mb):
        pltpu.sync_copy(mb, acc.at[ridx_v.at[j]], add=True)

    issue(0, mbuf0, sem0)

    def body(i, carry):
        issue(2 * i + 1, mbuf1, sem1)
        wait(mbuf0, sem0)
        scat(2 * i, mbuf0)

        @pl.when(i < (_NCH - 1) // 2 - 1)
        def _():
            issue(2 * i + 2, mbuf0, sem0)

        wait(mbuf1, sem1)
        scat(2 * i + 1, mbuf1)
        return carry

    lax.fori_loop(0, (_NCH - 1) // 2, body, 0)
    issue(_NCH - 1, mbuf0, sem0)
    wait(mbuf0, sem0)
    scat(_NCH - 1, mbuf0)
    plsc.subcore_barrier()

    @pl.when(s < 10)
    def _drain():
        def dbody(k, carry):
            pltpu.sync_copy(acc.at[pl.ds(s * 1000 + k * 125, 125)], zbuf)
            pltpu.sync_copy(zbuf, out_hbm.at[c * 10 + s, pl.ds(k * 125, 125)])
            return carry

        lax.fori_loop(0, 8, dbody, 0)


def _sc_scatter(ridx3, msg, zeros):
    run = functools.partial(
        pl.kernel,
        out_type=jax.ShapeDtypeStruct((20, 1000, _MD), F32),
        mesh=_sc_mesh(),
        scratch_types=[
            pltpu.VMEM((_NCH, _CH), jnp.int32),
            pltpu.VMEM((_CH, _MD), F32),
            pltpu.VMEM((_CH, _MD), F32),
            pltpu.VMEM((125, _MD), F32),
            pltpu.VMEM_SHARED((_N, _MD), F32),
            pltpu.SemaphoreType.DMA,
            pltpu.SemaphoreType.DMA,
        ],
        compiler_params=pltpu.CompilerParams(use_tc_tiling_on_sc=False),
    )(_scatter_body)
    return run(ridx3, msg, zeros)


def _edge_mlp_body(xr_ref, xs_ref, w0r_ref, w0s_ref, b0_ref,
                   w1_ref, b1_ref, w2_ref, b2_ref, w3_ref, b3_ref,
                   w4_ref, b4_ref, w5_ref, b5_ref, out_ref):
    bf = jnp.bfloat16

    def dotb(a, w):
        return jnp.dot(a, w, preferred_element_type=F32)

    pr = xr_ref[...]
    ps = xs_ref[...]
    w0r = w0r_ref[...]
    w0s = w0s_ref[...]
    parts = []
    for k in range(8):
        hk = (dotb(pr[:, 16 * k:16 * (k + 1)], w0r)
              + dotb(ps[:, 16 * k:16 * (k + 1)], w0s))
        parts.append(hk)
    h = jnp.concatenate(parts, axis=0)
    h = jnp.maximum(h + b0_ref[...], 0.0)
    h = jnp.maximum(dotb(h, w1_ref[...]) + b1_ref[...], 0.0)
    h = jnp.maximum(dotb(h, w2_ref[...]) + b2_ref[...], 0.0)
    h = jnp.maximum(dotb(h, w3_ref[...]) + b3_ref[...], 0.0)
    h = jnp.maximum(dotb(h, w4_ref[...]) + b4_ref[...], 0.0)
    msg = dotb(h, w5_ref[...]) + b5_ref[...]
    half = _BLK // 2
    out_ref[...] = jnp.concatenate([msg[:half], msg[half:]], axis=1)


def _edge_mlp(xr, xs, w0r, w0s, b0, w1, b1, w2, b2, w3, b3, w4, b4, w5, b5):
    rblk = _BLK * _D // 128
    oblk = _BLK * _MD // 128
    full = lambda shape: pl.BlockSpec(shape, lambda i: (0, 0))
    return pl.pallas_call(
        _edge_mlp_body,
        grid=(_NBLK,),
        in_specs=[
            pl.BlockSpec((rblk, 128), lambda i: (i, 0)),
            pl.BlockSpec((rblk, 128), lambda i: (i, 0)),
            full(w0r.shape), full(w0s.shape), full(b0.shape),
            full(w1.shape), full(b1.shape), full(w2.shape), full(b2.shape),
            full(w3.shape), full(b3.shape), full(w4.shape), full(b4.shape),
            full(w5.shape), full(b5.shape),
        ],
        out_specs=pl.BlockSpec((oblk, 128), lambda i: (i, 0)),
        out_shape=jax.ShapeDtypeStruct((_E * _MD // 128, 128), F32),
        compiler_params=pltpu.CompilerParams(
            dimension_semantics=("arbitrary",)),
    )(xr, xs, w0r, w0s, b0, w1, b1, w2, b2, w3, b3, w4, b4, w5, b5)


def _node_mlp_body(x2_ref, parts_ref, w0x_ref, w0a_ref, b0_ref,
                   w1_ref, b1_ref, w2_ref, b2_ref, out_ref):
    dot = functools.partial(jnp.dot, preferred_element_type=F32)
    half = _N // 2
    aggr = parts_ref[0:half, :] + parts_ref[half:_N, :]
    h = dot(x2_ref[...], w0x_ref[...]) + dot(aggr, w0a_ref[...])
    h = jnp.maximum(h + b0_ref[...], 0.0)
    h = jnp.maximum(dot(h, w1_ref[...]) + b1_ref[...], 0.0)
    out_ref[...] = dot(h, w2_ref[...]) + b2_ref[...]


def _node_mlp(x2, parts, w0x, w0a, b0, w1, b1, w2, b2):
    return pl.pallas_call(
        _node_mlp_body,
        out_shape=jax.ShapeDtypeStruct((_N // 2, 4), F32),
    )(x2, parts, w0x, w0a, b0, w1, b1, w2, b2)


def kernel(x, edge_index, edge_attr,
           e_w0, e_b0, e_w1, e_b1, e_w2, e_b2, e_w3, e_b3, e_w4, e_b4,
           e_w5, e_b5, n_w0, n_b0, n_w1, n_b1, n_w2, n_b2):
    del edge_attr
    sender = edge_index[0]
    receiver = edge_index[1]

    m_of_t = jnp.arange(_BLK).reshape(2, _BLK // 2).T.reshape(-1)
    sigma = 8 * (m_of_t % (_BLK // 8)) + m_of_t // (_BLK // 8)

    ridx_g = receiver.reshape(_NW, _NCH, _CH)
    sidx_g = sender.reshape(_NW, _NCH, _CH)

    xpad = jnp.zeros((_N, _D), F32).at[:, :5].set(x)

    a, b, cmat = e_w0[:, 0:5], e_w0[:, 5:10], e_w0[:, 10:15]
    w0r = jnp.zeros((_D, 150), F32).at[0:5, :].set((a + cmat).T)
    w0s = jnp.zeros((_D, 150), F32).at[0:5, :].set((b - cmat).T)
    w5 = jnp.zeros((150, _MD), F32).at[:, 0:50].set(e_w5.T)
    b5 = jnp.zeros((1, _MD), F32).at[:, 0:50].set(e_b5)

    xr, xs = _sc_gather(ridx_g, sidx_g, xpad)
    xr = xr.reshape(_E * _D // 128, 128)
    xs = xs.reshape(_E * _D // 128, 128)
    tok = lax.convert_element_type(xr[0, 0] * 0.0, jnp.int32)
    ridx_t = (receiver.reshape(_NBLK, _BLK)[:, sigma]
              .reshape(_NW, _NCH, _CH)) + tok
    msg = _edge_mlp(xr, xs, w0r, w0s,
                    e_b0.reshape(1, -1),
                    e_w1.T, e_b1.reshape(1, -1),
                    e_w2.T, e_b2.reshape(1, -1),
                    e_w3.T, e_b3.reshape(1, -1),
                    e_w4.T, e_b4.reshape(1, -1),
                    w5, b5)

    zeros = jnp.zeros((125, _MD), F32)
    partials = _sc_scatter(ridx_t, msg.reshape(_E // _CH, _CH, _MD), zeros)
    partials = partials.reshape(_N, 128)

    def blockdiag2(w):
        r, c = w.shape
        return (jnp.zeros((2 * r, 2 * c), F32)
                .at[0:r, 0:c].set(w).at[r:2 * r, c:2 * c].set(w))

    w0x = n_w0[:, 0:5].T
    w0a = jnp.zeros((_MD, 100), F32).at[0:50, :].set(n_w0[:, 5:55].T)
    out2 = _node_mlp(x.reshape(_N // 2, 10), partials,
                     blockdiag2(w0x), blockdiag2(w0a),
                     jnp.tile(n_b0, 2).reshape(1, -1),
                     blockdiag2(n_w1.T), jnp.tile(n_b1, 2).reshape(1, -1),
                     blockdiag2(n_w2.T), jnp.tile(n_b2, 2).reshape(1, -1))
    return out2.reshape(_N, 2)
